# baseline (speedup 1.0000x reference)
import functools

import jax
import jax.numpy as jnp
from jax import lax
from jax.experimental import pallas as pl
from jax.experimental.pallas import tpu as pltpu
from jax.experimental.pallas import tpu_sc as plsc

N = 10000
E = 320000
EH = E // 2
D = 128
ED = 16
HID = 2 * D
DH = D // 2

NTILE = 16
NCORE = 2
NW = NTILE * NCORE

EPT = EH // NTILE
G1 = 40
GPC = 10
NG1 = EPT // G1
NCHK = NG1 // GPC
NP = 10240
NR = NP // NTILE

EPW = EH // NW
G2 = 40
NG2 = EPW // G2

_mesh = plsc.VectorSubcoreMesh(core_axis_name="c", subcore_axis_name="s")


_BE = 4000


def _edge_lin_body(ea_ref, we_ref, be_ref, out_ref):
    acc = jnp.dot(ea_ref[...], we_ref[...], preferred_element_type=jnp.float32)
    out_ref[...] = acc + be_ref[0]


def _edge_lin(edge_attr, W_e, b_e, half):
    off = half * (EH // _BE)
    return pl.pallas_call(
        _edge_lin_body,
        grid=(EH // _BE,),
        in_specs=[
            pl.BlockSpec((_BE, ED), lambda i: (i + off, 0)),
            pl.BlockSpec((ED, D), lambda i: (0, 0)),
            pl.BlockSpec((1, D), lambda i: (0, 0)),
        ],
        out_specs=pl.BlockSpec((_BE, D), lambda i: (i, 0)),
        out_shape=jax.ShapeDtypeStruct((EH, D), jnp.float32),
    )(edge_attr, W_e, b_e.reshape(1, D))


def _p1_body(first, x, e2, src3, dst3, tvec, acc_prev, acc_out,
             idx_s, idx_d, xrows, erows, orows, tv, accsh,
             se, sx, ss):
    c = lax.axis_index("c")
    s = lax.axis_index("s")
    pltpu.sync_copy(tvec, tv)
    if first:
        def zrow(j, carry):
            for k in range(D // 16):
                orows[0, j, pl.ds(k * 16, 16)] = jnp.zeros((16,), jnp.float32)
            return carry

        lax.fori_loop(0, G1, zrow, 0)
        for i in range(NR // G1):
            pltpu.async_copy(orows.at[0],
                             accsh.at[pl.ds(s * NR + i * G1, G1)], ss[0])
        for i in range(NR // G1):
            pltpu.make_async_copy(e2.at[pl.ds(0, G1)], orows.at[0],
                                  ss[0]).wait()
    else:
        pltpu.sync_copy(acc_prev.at[c, pl.ds(s * NR, NR)],
                        accsh.at[pl.ds(s * NR, NR)])
    plsc.subcore_barrier()
    tval = tv[...]

    def issue_loads(cc, gg, b):
        pltpu.async_copy(
            e2.at[pl.ds(s * EPT + cc * (GPC * G1) + gg * G1, G1)],
            erows.at[b], se[b])
        pltpu.async_copy(x.at[idx_s.at[gg]], xrows.at[b], sx[b])

    def wait_loads(b):
        pltpu.make_async_copy(e2.at[pl.ds(0, G1)], erows.at[b], se[b]).wait()
        pltpu.make_async_copy(e2.at[pl.ds(0, G1)], xrows.at[b], sx[b]).wait()

    def wait_scat(b):
        pltpu.make_async_copy(e2.at[pl.ds(0, G1)], orows.at[b], ss[b]).wait()

    def compute(gg, b):
        def half(hoff):
            def edge(j, carry2):
                for k in range(DH // 16):
                    sl = pl.ds(hoff + k * 16, 16)
                    m = jnp.maximum(xrows[b, j, sl] + erows[b, j, sl], 0.0) + 1e-7
                    w = jnp.exp(m * tval)
                    orows[b, j, pl.ds(k * 16, 16)] = w * m
                    orows[b, j, pl.ds(DH + k * 16, 16)] = w
                return carry2

            lax.fori_loop(0, G1, edge, 0)

        @pl.when(c == 0)
        def _():
            half(0)

        @pl.when(c != 0)
        def _():
            half(DH)

        pltpu.async_copy(orows.at[b], accsh.at[idx_d.at[gg]], ss[b], add=True)

    def chunk(cc, carry0):
        pltpu.sync_copy(src3.at[s, cc], idx_s)
        pltpu.sync_copy(dst3.at[s, cc], idx_d)
        issue_loads(cc, 0, 0)

        def pair(p, carry):
            for b in (0, 1):
                g = 2 * p + b

                @pl.when(g + 1 < GPC)
                def _():
                    issue_loads(cc, g + 1, 1 - b)

                wait_loads(b)

                @pl.when(g >= 2)
                def _():
                    wait_scat(b)

                compute(g, b)
            return carry

        lax.fori_loop(0, GPC // 2, pair, 0)
        wait_scat(0)
        wait_scat(1)
        return carry0

    lax.fori_loop(0, NCHK, chunk, 0)
    plsc.subcore_barrier()
    pltpu.sync_copy(accsh.at[pl.ds(s * NR, NR)],
                    acc_out.at[c, pl.ds(s * NR, NR)])


_P1_SCRATCH = [
    pltpu.VMEM((GPC, G1), jnp.int32),
    pltpu.VMEM((GPC, G1), jnp.int32),
    pltpu.VMEM((2, G1, D), jnp.float32),
    pltpu.VMEM((2, G1, D), jnp.float32),
    pltpu.VMEM((2, G1, D), jnp.float32),
    pltpu.VMEM((16,), jnp.float32),
    pltpu.VMEM_SHARED((NP, D), jnp.float32),
    [pltpu.SemaphoreType.DMA, pltpu.SemaphoreType.DMA],
    [pltpu.SemaphoreType.DMA, pltpu.SemaphoreType.DMA],
    [pltpu.SemaphoreType.DMA, pltpu.SemaphoreType.DMA],
]


@functools.partial(
    pl.kernel,
    out_type=jax.ShapeDtypeStruct((2, NP, D), jnp.float32),
    mesh=_mesh,
    scratch_types=_P1_SCRATCH,
)
def _pass1a(x, e2, src3, dst3, tvec, acc_out, *scratch):
    _p1_body(True, x, e2, src3, dst3, tvec, None, acc_out, *scratch)


@functools.partial(
    pl.kernel,
    out_type=jax.ShapeDtypeStruct((2, NP, D), jnp.float32),
    mesh=_mesh,
    scratch_types=_P1_SCRATCH,
)
def _pass1b(x, e2, src3, dst3, tvec, acc_prev, acc_out, *scratch):
    _p1_body(False, x, e2, src3, dst3, tvec, acc_prev, acc_out, *scratch)


def _node_body(acc_ref, x_ref, w1_ref, b1_ref, bng_ref, bnb_ref, w2_ref,
               b2_ref, lng_ref, lnb_ref, emw_ref, emb_ref,
               ox_ref, p_ref, q_ref):
    num = jnp.concatenate([acc_ref[0, :N, :DH], acc_ref[1, :N, :DH]], axis=1)
    den = jnp.concatenate([acc_ref[0, :N, DH:], acc_ref[1, :N, DH:]], axis=1)
    x = x_ref[...]
    out = num / (den + 1e-16) + x
    h = jnp.dot(out, w1_ref[...], preferred_element_type=jnp.float32) + b1_ref[0]
    mu = jnp.mean(h, axis=0, keepdims=True)
    var = jnp.mean((h - mu) ** 2, axis=0, keepdims=True)
    h = (h - mu) / jnp.sqrt(var + 1e-5) * bng_ref[0] + bnb_ref[0]
    h = jnp.maximum(h, 0.0)
    h = jnp.dot(h, w2_ref[...], preferred_element_type=jnp.float32) + b2_ref[0]
    mu2 = jnp.mean(h, axis=1, keepdims=True)
    var2 = jnp.mean((h - mu2) ** 2, axis=1, keepdims=True)
    h = (h - mu2) / jnp.sqrt(var2 + 1e-5) * lng_ref[0] + lnb_ref[0]
    ox = jnp.where(h > 0.0, h, jnp.exp(jnp.minimum(h, 0.0)) - 1.0)
    ox_ref[...] = ox
    hb = 0.5 * emb_ref[0]
    p_ref[...] = jnp.dot(ox, emw_ref[...][:D], preferred_element_type=jnp.float32) + hb
    q_ref[...] = jnp.dot(ox, emw_ref[...][D:], preferred_element_type=jnp.float32) + hb


def _node_dense(acc, x, W1, b1, bn_g, bn_b, W2, b2, ln_g, ln_b, em_W, em_b):
    return pl.pallas_call(
        _node_body,
        out_shape=[
            jax.ShapeDtypeStruct((N, D), jnp.float32),
            jax.ShapeDtypeStruct((N, D), jnp.float32),
            jax.ShapeDtypeStruct((N, D), jnp.float32),
        ],
    )(acc, x, W1, b1.reshape(1, HID), bn_g.reshape(1, HID),
      bn_b.reshape(1, HID), W2, b2.reshape(1, D), ln_g.reshape(1, D),
      ln_b.reshape(1, D), em_W, em_b.reshape(1, D))


def _p2_body(p_t, q_t, src4, dst4, s_out,
             idx_s, idx_d, prow, qrow, sbuf, sp, sq, ss):
    c = lax.axis_index("c")
    s = lax.axis_index("s")
    w = s * NCORE + c
    pltpu.sync_copy(src4.at[w], idx_s)
    pltpu.sync_copy(dst4.at[w], idx_d)

    def issue_loads(g, b):
        pltpu.async_copy(p_t.at[idx_s.at[g]], prow.at[b], sp[b])
        pltpu.async_copy(q_t.at[idx_d.at[g]], qrow.at[b], sq[b])

    def wait_loads(b):
        pltpu.make_async_copy(p_t.at[pl.ds(0, G2)], prow.at[b], sp[b]).wait()
        pltpu.make_async_copy(p_t.at[pl.ds(0, G2)], qrow.at[b], sq[b]).wait()

    def wait_store(b):
        pltpu.make_async_copy(p_t.at[pl.ds(0, G2)], sbuf.at[b], ss[b]).wait()

    def iteration(g, b):
        @pl.when(g + 1 < NG2)
        def _():
            issue_loads(g + 1, 1 - b)

        wait_loads(b)

        @pl.when(g >= 2)
        def _():
            wait_store(b)

        def edge(j, carry2):
            for k in range(D // 16):
                sl = pl.ds(k * 16, 16)
                sbuf[b, j, sl] = prow[b, j, sl] + qrow[b, j, sl]
            return carry2

        lax.fori_loop(0, G2, edge, 0)
        pltpu.async_copy(sbuf.at[b], s_out.at[pl.ds(w * EPW + g * G2, G2)],
                         ss[b])

    issue_loads(0, 0)

    def pair(p, carry):
        for b in (0, 1):
            iteration(2 * p + b, b)
        return carry

    lax.fori_loop(0, NG2 // 2, pair, 0)
    iteration(NG2 - 1, 0)
    wait_store(1)
    wait_store(0)


@functools.partial(
    pl.kernel,
    out_type=jax.ShapeDtypeStruct((EH, D), jnp.float32),
    mesh=_mesh,
    scratch_types=[
        pltpu.VMEM((NG2, G2), jnp.int32),
        pltpu.VMEM((NG2, G2), jnp.int32),
        pltpu.VMEM((2, G2, D), jnp.float32),
        pltpu.VMEM((2, G2, D), jnp.float32),
        pltpu.VMEM((2, G2, D), jnp.float32),
        [pltpu.SemaphoreType.DMA, pltpu.SemaphoreType.DMA],
        [pltpu.SemaphoreType.DMA, pltpu.SemaphoreType.DMA],
        [pltpu.SemaphoreType.DMA, pltpu.SemaphoreType.DMA],
    ],
)
def _pass2(*refs):
    _p2_body(*refs)


_BF = 2000
_INV_SQRT2 = 0.7071067811865476


def _gelu_ln_first_body(s_ref, g_ref, b_ref, out_ref):
    v = s_ref[...]
    g = 0.5 * v * (1.0 + lax.erf(v * _INV_SQRT2))
    mu = jnp.mean(g, axis=1, keepdims=True)
    var = jnp.mean((g - mu) ** 2, axis=1, keepdims=True)
    out_ref[...] = (g - mu) / jnp.sqrt(var + 1e-5) * g_ref[0] + b_ref[0]


def _gelu_ln_second_body(s_ref, g_ref, b_ref, prev_ref, out_ref):
    _gelu_ln_first_body(s_ref, g_ref, b_ref, out_ref)


def _gelu_ln_first(S, eln_g, eln_b):
    return pl.pallas_call(
        _gelu_ln_first_body,
        grid=(EH // _BF,),
        in_specs=[
            pl.BlockSpec((_BF, D), lambda i: (i, 0)),
            pl.BlockSpec((1, D), lambda i: (0, 0)),
            pl.BlockSpec((1, D), lambda i: (0, 0)),
        ],
        out_specs=pl.BlockSpec((_BF, D), lambda i: (i, 0)),
        out_shape=jax.ShapeDtypeStruct((E, D), jnp.float32),
    )(S, eln_g.reshape(1, D), eln_b.reshape(1, D))


def _gelu_ln_second(S, eln_g, eln_b, prev):
    nblk = EH // _BF
    return pl.pallas_call(
        _gelu_ln_second_body,
        grid=(nblk,),
        in_specs=[
            pl.BlockSpec((_BF, D), lambda i: (i, 0)),
            pl.BlockSpec((1, D), lambda i: (0, 0)),
            pl.BlockSpec((1, D), lambda i: (0, 0)),
            pl.BlockSpec(memory_space=pl.ANY),
        ],
        out_specs=pl.BlockSpec((_BF, D), lambda i: (i + nblk, 0)),
        out_shape=jax.ShapeDtypeStruct((E, D), jnp.float32),
        input_output_aliases={3: 0},
    )(S, eln_g.reshape(1, D), eln_b.reshape(1, D), prev)


def kernel(x, edge_index, edge_attr, W_e, b_e, t, W1, b1, bn_g, bn_b, W2, b2,
           ln_g, ln_b, em_W, em_b, eln_g, eln_b):
    src = edge_index[0]
    dst = edge_index[1]
    srcr = src.reshape(2, NTILE, NCHK, GPC, G1)
    dstr = dst.reshape(2, NTILE, NCHK, GPC, G1)
    src4 = src.reshape(2, NW, NG2, G2)
    dst4 = dst.reshape(2, NW, NG2, G2)
    tvec = jnp.full((16,), t, dtype=jnp.float32)

    e2a = _edge_lin(edge_attr, W_e, b_e, 0)
    e2b = _edge_lin(edge_attr, W_e, b_e, 1)
    acc1 = _pass1a(x, e2a, srcr[0], dstr[0], tvec)
    acc = _pass1b(x, e2b, srcr[1], dstr[1], tvec, acc1)
    out_x, p_t, q_t = _node_dense(acc, x, W1, b1, bn_g, bn_b, W2, b2,
                                  ln_g, ln_b, em_W, em_b)
    s_a = _pass2(p_t, q_t, src4[0], dst4[0])
    s_b = _pass2(p_t, q_t, src4[1], dst4[1])
    oe_a = _gelu_ln_first(s_a, eln_g, eln_b)
    out_e = _gelu_ln_second(s_b, eln_g, eln_b, oe_a)
    return (out_x, out_e)


# speedup vs baseline: 1.0130x; 1.0130x over previous
import functools

import jax
import jax.numpy as jnp
from jax import lax
from jax.experimental import pallas as pl
from jax.experimental.pallas import tpu as pltpu
from jax.experimental.pallas import tpu_sc as plsc

N = 10000
E = 320000
EH = E // 2
D = 128
ED = 16
HID = 2 * D
DH = D // 2

NTILE = 16
NCORE = 2
NW = NTILE * NCORE

EPT = EH // NTILE
G1 = 40
GPC = 10
NG1 = EPT // G1
NCHK = NG1 // GPC
NP = 10240
NR = NP // NTILE

EPW = EH // NW
G2 = 40
NG2 = EPW // G2

_mesh = plsc.VectorSubcoreMesh(core_axis_name="c", subcore_axis_name="s")


_BE = 4000


def _edge_lin_body(ea_ref, we_ref, be_ref, out_ref):
    acc = jnp.dot(ea_ref[...], we_ref[...], preferred_element_type=jnp.float32)
    out_ref[...] = acc + be_ref[0]


def _edge_lin(edge_attr, W_e, b_e):
    return pl.pallas_call(
        _edge_lin_body,
        grid=(EH // _BE,),
        in_specs=[
            pl.BlockSpec((_BE, ED), lambda i: (i, 0)),
            pl.BlockSpec((ED, D), lambda i: (0, 0)),
            pl.BlockSpec((1, D), lambda i: (0, 0)),
        ],
        out_specs=pl.BlockSpec((_BE, D), lambda i: (i, 0)),
        out_shape=jax.ShapeDtypeStruct((EH, D), jnp.float32),
    )(edge_attr, W_e, b_e.reshape(1, D))


def _p1_body(first, x, e2, src3, dst3, tvec, acc_prev, acc_out,
             idx_s, idx_d, xrows, erows, orows, tv, accsh,
             se, sx, ss):
    c = lax.axis_index("c")
    s = lax.axis_index("s")
    pltpu.sync_copy(tvec, tv)
    if first:
        def zrow(j, carry):
            for k in range(D // 16):
                orows[0, j, pl.ds(k * 16, 16)] = jnp.zeros((16,), jnp.float32)
            return carry

        lax.fori_loop(0, G1, zrow, 0)
        for i in range(NR // G1):
            pltpu.async_copy(orows.at[0],
                             accsh.at[pl.ds(s * NR + i * G1, G1)], ss[0])
        for i in range(NR // G1):
            pltpu.make_async_copy(e2.at[pl.ds(0, G1)], orows.at[0],
                                  ss[0]).wait()
    else:
        pltpu.sync_copy(acc_prev.at[c, pl.ds(s * NR, NR)],
                        accsh.at[pl.ds(s * NR, NR)])
    plsc.subcore_barrier()
    tval = tv[...]

    def issue_loads(cc, gg, b):
        pltpu.async_copy(
            e2.at[pl.ds(s * EPT + cc * (GPC * G1) + gg * G1, G1)],
            erows.at[b], se[b])
        pltpu.async_copy(x.at[idx_s.at[gg]], xrows.at[b], sx[b])

    def wait_loads(b):
        pltpu.make_async_copy(e2.at[pl.ds(0, G1)], erows.at[b], se[b]).wait()
        pltpu.make_async_copy(e2.at[pl.ds(0, G1)], xrows.at[b], sx[b]).wait()

    def wait_scat(b):
        pltpu.make_async_copy(e2.at[pl.ds(0, G1)], orows.at[b], ss[b]).wait()

    def compute(gg, b):
        def half(hoff):
            def edge(j, carry2):
                for k in range(DH // 16):
                    sl = pl.ds(hoff + k * 16, 16)
                    m = jnp.maximum(xrows[b, j, sl] + erows[b, j, sl], 0.0) + 1e-7
                    w = jnp.exp(m * tval)
                    orows[b, j, pl.ds(k * 16, 16)] = w * m
                    orows[b, j, pl.ds(DH + k * 16, 16)] = w
                return carry2

            lax.fori_loop(0, G1, edge, 0)

        @pl.when(c == 0)
        def _():
            half(0)

        @pl.when(c != 0)
        def _():
            half(DH)

        pltpu.async_copy(orows.at[b], accsh.at[idx_d.at[gg]], ss[b], add=True)

    def chunk(cc, carry0):
        pltpu.sync_copy(src3.at[s, cc], idx_s)
        pltpu.sync_copy(dst3.at[s, cc], idx_d)
        issue_loads(cc, 0, 0)

        def pair(p, carry):
            for b in (0, 1):
                g = 2 * p + b

                @pl.when(g + 1 < GPC)
                def _():
                    issue_loads(cc, g + 1, 1 - b)

                wait_loads(b)

                @pl.when(g >= 2)
                def _():
                    wait_scat(b)

                compute(g, b)
            return carry

        lax.fori_loop(0, GPC // 2, pair, 0)
        wait_scat(0)
        wait_scat(1)
        return carry0

    lax.fori_loop(0, NCHK, chunk, 0)
    plsc.subcore_barrier()
    pltpu.sync_copy(accsh.at[pl.ds(s * NR, NR)],
                    acc_out.at[c, pl.ds(s * NR, NR)])


_P1_SCRATCH = [
    pltpu.VMEM((GPC, G1), jnp.int32),
    pltpu.VMEM((GPC, G1), jnp.int32),
    pltpu.VMEM((2, G1, D), jnp.float32),
    pltpu.VMEM((2, G1, D), jnp.float32),
    pltpu.VMEM((2, G1, D), jnp.float32),
    pltpu.VMEM((16,), jnp.float32),
    pltpu.VMEM_SHARED((NP, D), jnp.float32),
    [pltpu.SemaphoreType.DMA, pltpu.SemaphoreType.DMA],
    [pltpu.SemaphoreType.DMA, pltpu.SemaphoreType.DMA],
    [pltpu.SemaphoreType.DMA, pltpu.SemaphoreType.DMA],
]


@functools.partial(
    pl.kernel,
    out_type=jax.ShapeDtypeStruct((2, NP, D), jnp.float32),
    mesh=_mesh,
    scratch_types=_P1_SCRATCH,
)
def _pass1a(x, e2, src3, dst3, tvec, acc_out, *scratch):
    _p1_body(True, x, e2, src3, dst3, tvec, None, acc_out, *scratch)


@functools.partial(
    pl.kernel,
    out_type=jax.ShapeDtypeStruct((2, NP, D), jnp.float32),
    mesh=_mesh,
    scratch_types=_P1_SCRATCH,
)
def _pass1b(x, e2, src3, dst3, tvec, acc_prev, acc_out, *scratch):
    _p1_body(False, x, e2, src3, dst3, tvec, acc_prev, acc_out, *scratch)


def _node_body(acc_ref, x_ref, w1_ref, b1_ref, bng_ref, bnb_ref, w2_ref,
               b2_ref, lng_ref, lnb_ref, emw_ref, emb_ref,
               ox_ref, p_ref, q_ref):
    num = jnp.concatenate([acc_ref[0, :N, :DH], acc_ref[1, :N, :DH]], axis=1)
    den = jnp.concatenate([acc_ref[0, :N, DH:], acc_ref[1, :N, DH:]], axis=1)
    x = x_ref[...]
    out = num / (den + 1e-16) + x
    h = jnp.dot(out, w1_ref[...], preferred_element_type=jnp.float32) + b1_ref[0]
    mu = jnp.mean(h, axis=0, keepdims=True)
    var = jnp.mean((h - mu) ** 2, axis=0, keepdims=True)
    h = (h - mu) / jnp.sqrt(var + 1e-5) * bng_ref[0] + bnb_ref[0]
    h = jnp.maximum(h, 0.0)
    h = jnp.dot(h, w2_ref[...], preferred_element_type=jnp.float32) + b2_ref[0]
    mu2 = jnp.mean(h, axis=1, keepdims=True)
    var2 = jnp.mean((h - mu2) ** 2, axis=1, keepdims=True)
    h = (h - mu2) / jnp.sqrt(var2 + 1e-5) * lng_ref[0] + lnb_ref[0]
    ox = jnp.where(h > 0.0, h, jnp.exp(jnp.minimum(h, 0.0)) - 1.0)
    ox_ref[...] = ox
    hb = 0.5 * emb_ref[0]
    p_ref[...] = jnp.dot(ox, emw_ref[...][:D], preferred_element_type=jnp.float32) + hb
    q_ref[...] = jnp.dot(ox, emw_ref[...][D:], preferred_element_type=jnp.float32) + hb


def _node_dense(acc, x, W1, b1, bn_g, bn_b, W2, b2, ln_g, ln_b, em_W, em_b):
    return pl.pallas_call(
        _node_body,
        out_shape=[
            jax.ShapeDtypeStruct((N, D), jnp.float32),
            jax.ShapeDtypeStruct((N, D), jnp.float32),
            jax.ShapeDtypeStruct((N, D), jnp.float32),
        ],
    )(acc, x, W1, b1.reshape(1, HID), bn_g.reshape(1, HID),
      bn_b.reshape(1, HID), W2, b2.reshape(1, D), ln_g.reshape(1, D),
      ln_b.reshape(1, D), em_W, em_b.reshape(1, D))


def _p2_body(p_t, q_t, src4, dst4, s_out,
             idx_s, idx_d, prow, qrow, sbuf, sp, sq, ss):
    c = lax.axis_index("c")
    s = lax.axis_index("s")
    w = s * NCORE + c
    pltpu.sync_copy(src4.at[w], idx_s)
    pltpu.sync_copy(dst4.at[w], idx_d)

    def issue_loads(g, b):
        pltpu.async_copy(p_t.at[idx_s.at[g]], prow.at[b], sp[b])
        pltpu.async_copy(q_t.at[idx_d.at[g]], qrow.at[b], sq[b])

    def wait_loads(b):
        pltpu.make_async_copy(p_t.at[pl.ds(0, G2)], prow.at[b], sp[b]).wait()
        pltpu.make_async_copy(p_t.at[pl.ds(0, G2)], qrow.at[b], sq[b]).wait()

    def wait_store(b):
        pltpu.make_async_copy(p_t.at[pl.ds(0, G2)], sbuf.at[b], ss[b]).wait()

    def iteration(g, b):
        @pl.when(g + 1 < NG2)
        def _():
            issue_loads(g + 1, 1 - b)

        wait_loads(b)

        @pl.when(g >= 2)
        def _():
            wait_store(b)

        def edge(j, carry2):
            for k in range(D // 16):
                sl = pl.ds(k * 16, 16)
                sbuf[b, j, sl] = prow[b, j, sl] + qrow[b, j, sl]
            return carry2

        lax.fori_loop(0, G2, edge, 0)
        pltpu.async_copy(sbuf.at[b], s_out.at[pl.ds(w * EPW + g * G2, G2)],
                         ss[b])

    issue_loads(0, 0)

    def pair(p, carry):
        for b in (0, 1):
            iteration(2 * p + b, b)
        return carry

    lax.fori_loop(0, NG2 // 2, pair, 0)
    iteration(NG2 - 1, 0)
    wait_store(1)
    wait_store(0)


@functools.partial(
    pl.kernel,
    out_type=jax.ShapeDtypeStruct((EH, D), jnp.float32),
    mesh=_mesh,
    scratch_types=[
        pltpu.VMEM((NG2, G2), jnp.int32),
        pltpu.VMEM((NG2, G2), jnp.int32),
        pltpu.VMEM((2, G2, D), jnp.float32),
        pltpu.VMEM((2, G2, D), jnp.float32),
        pltpu.VMEM((2, G2, D), jnp.float32),
        [pltpu.SemaphoreType.DMA, pltpu.SemaphoreType.DMA],
        [pltpu.SemaphoreType.DMA, pltpu.SemaphoreType.DMA],
        [pltpu.SemaphoreType.DMA, pltpu.SemaphoreType.DMA],
    ],
)
def _pass2(*refs):
    _p2_body(*refs)


_BF = 2000
_INV_SQRT2 = 0.7071067811865476


def _gelu_ln_first_body(s_ref, g_ref, b_ref, out_ref):
    v = s_ref[...]
    g = 0.5 * v * (1.0 + lax.erf(v * _INV_SQRT2))
    mu = jnp.mean(g, axis=1, keepdims=True)
    var = jnp.mean((g - mu) ** 2, axis=1, keepdims=True)
    out_ref[...] = (g - mu) / jnp.sqrt(var + 1e-5) * g_ref[0] + b_ref[0]


def _gelu_ln_second_body(s_ref, g_ref, b_ref, prev_ref, out_ref):
    _gelu_ln_first_body(s_ref, g_ref, b_ref, out_ref)


def _gelu_ln_first(S, eln_g, eln_b):
    return pl.pallas_call(
        _gelu_ln_first_body,
        grid=(EH // _BF,),
        in_specs=[
            pl.BlockSpec((_BF, D), lambda i: (i, 0)),
            pl.BlockSpec((1, D), lambda i: (0, 0)),
            pl.BlockSpec((1, D), lambda i: (0, 0)),
        ],
        out_specs=pl.BlockSpec((_BF, D), lambda i: (i, 0)),
        out_shape=jax.ShapeDtypeStruct((E, D), jnp.float32),
    )(S, eln_g.reshape(1, D), eln_b.reshape(1, D))


def _gelu_ln_second(S, eln_g, eln_b, prev):
    nblk = EH // _BF
    return pl.pallas_call(
        _gelu_ln_second_body,
        grid=(nblk,),
        in_specs=[
            pl.BlockSpec((_BF, D), lambda i: (i, 0)),
            pl.BlockSpec((1, D), lambda i: (0, 0)),
            pl.BlockSpec((1, D), lambda i: (0, 0)),
            pl.BlockSpec(memory_space=pl.ANY),
        ],
        out_specs=pl.BlockSpec((_BF, D), lambda i: (i + nblk, 0)),
        out_shape=jax.ShapeDtypeStruct((E, D), jnp.float32),
        input_output_aliases={3: 0},
    )(S, eln_g.reshape(1, D), eln_b.reshape(1, D), prev)


def kernel(x, edge_index, edge_attr, W_e, b_e, t, W1, b1, bn_g, bn_b, W2, b2,
           ln_g, ln_b, em_W, em_b, eln_g, eln_b):
    src = edge_index[0]
    dst = edge_index[1]
    srcr = src.reshape(2, NTILE, NCHK, GPC, G1)
    dstr = dst.reshape(2, NTILE, NCHK, GPC, G1)
    src4 = src.reshape(2, NW, NG2, G2)
    dst4 = dst.reshape(2, NW, NG2, G2)
    tvec = jnp.full((16,), t, dtype=jnp.float32)

    e2a = _edge_lin(edge_attr[:EH], W_e, b_e)
    e2b = _edge_lin(edge_attr[EH:], W_e, b_e)
    acc1 = _pass1a(x, e2a, srcr[0], dstr[0], tvec)
    acc = _pass1b(x, e2b, srcr[1], dstr[1], tvec, acc1)
    out_x, p_t, q_t = _node_dense(acc, x, W1, b1, bn_g, bn_b, W2, b2,
                                  ln_g, ln_b, em_W, em_b)
    s_a = _pass2(p_t, q_t, src4[0], dst4[0])
    s_b = _pass2(p_t, q_t, src4[1], dst4[1])
    oe_a = _gelu_ln_first(s_a, eln_g, eln_b)
    out_e = _gelu_ln_second(s_b, eln_g, eln_b, oe_a)
    return (out_x, out_e)
